# baseline (speedup 1.0000x reference)
import jax
import jax.numpy as jnp
from jax.experimental import pallas as pl
from jax.experimental.pallas import tpu as pltpu

_N = 4096
_NB = 4
_T = 256
_NT = _N // _T
_K = 4
_GJ = _NT // _K


def _focal_body(sched_ref, batch_ref, *refs):
    pred_refs = refs[:_K]
    y_refs = refs[_K : 2 * _K]
    out_ref = refs[2 * _K]
    inv_ref = refs[2 * _K + 1]
    i = pl.program_id(0)
    j = pl.program_id(1)
    cstart = sched_ref[0, i]
    cnum = sched_ref[1, i]

    @pl.when((i == 0) & (j == 0))
    def _():
        out_ref[...] = jnp.zeros_like(out_ref)
        b_all = batch_ref[0, :]
        for b in range(_NB):
            cnt = jnp.sum((b_all == b).astype(jnp.float32))
            inv_ref[b] = 1.0 / jnp.maximum(cnt, 1.0)

    brow = batch_ref[0, pl.ds(i * _T, _T)]
    wrow = jnp.zeros((_T,), jnp.float32)
    for b in range(_NB):
        wrow = wrow + (brow == b).astype(jnp.float32) * inv_ref[b]

    for k in range(_K):
        @pl.when(j * _K + k < cnum)
        def _(k=k):
            ci = cstart + j * _K + k
            bcol = batch_ref[0, pl.ds(ci * _T, _T)]
            wcol = jnp.zeros((_T,), jnp.float32)
            for b in range(_NB):
                wcol = wcol + (bcol == b).astype(jnp.float32) * inv_ref[b]

            x = pred_refs[k][...]
            log_p = jax.nn.log_sigmoid(x)
            p = jnp.exp(log_p)
            omp = 1.0 - p
            loss = -jnp.where(
                y_refs[k][...] != 0, omp * omp * log_p, p * p * (log_p - x)
            )

            eq = brow[:, None] == bcol[None, :]
            wmat = wrow[:, None] * wcol[None, :]
            contrib = jnp.sum(jnp.where(eq, loss * wmat, 0.0))
            out_ref[...] += contrib.reshape(1, 1)


def _make_schedule(batch):
    ids = jnp.arange(_NB, dtype=batch.dtype)
    seg_s = jnp.searchsorted(batch, ids, side="left").astype(jnp.int32)
    seg_e = jnp.searchsorted(batch, ids, side="right").astype(jnp.int32)
    first = batch[:: _T].astype(jnp.int32)
    last = batch[_T - 1 :: _T].astype(jnp.int32)
    cstart = seg_s[first] // _T
    cend = (seg_e[last] - 1) // _T
    return jnp.stack([cstart, cend - cstart + 1])


def kernel(y_seg_pred, y_seg, batch):
    batch = batch.astype(jnp.int32)
    sched = _make_schedule(batch)
    batch2d = batch.reshape(1, _N)

    def tile_spec(k):
        def idx(i, j, s, k=k):
            ci = s[0, i] + jnp.minimum(j * _K + k, s[1, i] - 1)
            return (i, ci)

        return pl.BlockSpec((_T, _T), idx)

    total = pl.pallas_call(
        _focal_body,
        grid_spec=pltpu.PrefetchScalarGridSpec(
            num_scalar_prefetch=1,
            grid=(_NT, _GJ),
            in_specs=[pl.BlockSpec((1, _N), lambda i, j, s: (0, 0))]
            + [tile_spec(k) for k in range(_K)]
            + [tile_spec(k) for k in range(_K)],
            out_specs=pl.BlockSpec((1, 1), lambda i, j, s: (0, 0)),
            scratch_shapes=[pltpu.SMEM((_NB,), jnp.float32)],
        ),
        out_shape=jax.ShapeDtypeStruct((1, 1), jnp.float32),
    )(sched, batch2d, *([y_seg_pred] * _K), *([y_seg] * _K))
    batch_size = (batch[-1] + 1).astype(jnp.float32)
    return total[0, 0] / batch_size


# speedup vs baseline: 1.6540x; 1.6540x over previous
import jax
import jax.numpy as jnp
from jax.experimental import pallas as pl
from jax.experimental.pallas import tpu as pltpu

_N = 4096
_NB = 4
_T = 256
_NT = _N // _T
_K = 4
_NTILES = _NT * _NT
_G = _NTILES // _K


def _focal_body(sched_ref, batch_ref, *refs):
    pred_refs = refs[:_K]
    y_refs = refs[_K : 2 * _K]
    out_ref = refs[2 * _K]
    inv_ref = refs[2 * _K + 1]
    g = pl.program_id(0)
    m_valid = sched_ref[2, 0]

    @pl.when(g == 0)
    def _():
        out_ref[...] = jnp.zeros_like(out_ref)
        b_all = batch_ref[0, :]
        for b in range(_NB):
            cnt = jnp.sum((b_all == b).astype(jnp.float32))
            inv_ref[b] = 1.0 / jnp.maximum(cnt, 1.0)

    for k in range(_K):
        @pl.when(g * _K + k < m_valid)
        def _(k=k):
            ri = sched_ref[0, g * _K + k]
            ci = sched_ref[1, g * _K + k]
            brow = batch_ref[0, pl.ds(ri * _T, _T)]
            bcol = batch_ref[0, pl.ds(ci * _T, _T)]
            wrow = jnp.zeros((_T,), jnp.float32)
            wcol = jnp.zeros((_T,), jnp.float32)
            for b in range(_NB):
                wrow = wrow + (brow == b).astype(jnp.float32) * inv_ref[b]
                wcol = wcol + (bcol == b).astype(jnp.float32) * inv_ref[b]

            x = pred_refs[k][...]
            log_p = jax.nn.log_sigmoid(x)
            p = jnp.exp(log_p)
            omp = 1.0 - p
            loss = -jnp.where(
                y_refs[k][...] != 0, omp * omp * log_p, p * p * (log_p - x)
            )

            eq = brow[:, None] == bcol[None, :]
            wmat = wrow[:, None] * wcol[None, :]
            contrib = jnp.sum(jnp.where(eq, loss * wmat, 0.0))
            out_ref[...] += contrib.reshape(1, 1)


def _make_schedule(batch):
    first = batch[:: _T]
    last = batch[_T - 1 :: _T]
    needed = (first[:, None] <= last[None, :]) & (first[None, :] <= last[:, None])
    flat = needed.reshape(-1)
    m = jnp.sum(flat.astype(jnp.int32))

    pos = jnp.cumsum(flat.astype(jnp.int32)) - 1
    tile_ids = jnp.arange(_NTILES, dtype=jnp.int32)
    perm = jnp.zeros(_NTILES, jnp.int32).at[
        jnp.where(flat, pos, _NTILES)
    ].set(tile_ids, mode="drop")
    idx = jnp.where(tile_ids < m, perm, perm[m - 1])
    sched = jnp.stack(
        [idx // _NT, idx % _NT, jnp.full((_NTILES,), m, dtype=jnp.int32)]
    )
    return sched


def kernel(y_seg_pred, y_seg, batch):
    batch = batch.astype(jnp.int32)
    sched = _make_schedule(batch)
    batch2d = batch.reshape(1, _N)

    def tile_spec(k):
        return pl.BlockSpec(
            (_T, _T), lambda g, s, k=k: (s[0, g * _K + k], s[1, g * _K + k])
        )

    total = pl.pallas_call(
        _focal_body,
        grid_spec=pltpu.PrefetchScalarGridSpec(
            num_scalar_prefetch=1,
            grid=(_G,),
            in_specs=[pl.BlockSpec((1, _N), lambda g, s: (0, 0))]
            + [tile_spec(k) for k in range(_K)]
            + [tile_spec(k) for k in range(_K)],
            out_specs=pl.BlockSpec((1, 1), lambda g, s: (0, 0)),
            scratch_shapes=[pltpu.SMEM((_NB,), jnp.float32)],
        ),
        out_shape=jax.ShapeDtypeStruct((1, 1), jnp.float32),
    )(sched, batch2d, *([y_seg_pred] * _K), *([y_seg] * _K))
    batch_size = (batch[-1] + 1).astype(jnp.float32)
    return total[0, 0] / batch_size


# speedup vs baseline: 1.8799x; 1.1365x over previous
import jax
import jax.numpy as jnp
from jax.experimental import pallas as pl
from jax.experimental.pallas import tpu as pltpu

_N = 4096
_NB = 4
_T = 256
_NT = _N // _T
_K = 4
_NTILES = _NT * _NT
_G = _NTILES // _K


def _focal_body(sched_ref, batch_ref, *refs):
    pred_refs = refs[:_K]
    y_refs = refs[_K : 2 * _K]
    out_ref = refs[2 * _K]
    inv_ref = refs[2 * _K + 1]
    acc_ref = refs[2 * _K + 2]
    g = pl.program_id(0)
    m_valid = sched_ref[2, 0]

    @pl.when(g == 0)
    def _():
        acc_ref[...] = jnp.zeros_like(acc_ref)
        b_all = batch_ref[0, :]
        for b in range(_NB):
            cnt = jnp.sum((b_all == b).astype(jnp.float32))
            inv_ref[b] = 1.0 / jnp.maximum(cnt, 1.0)

    for k in range(_K):
        @pl.when(g * _K + k < m_valid)
        def _(k=k):
            ri = sched_ref[0, g * _K + k]
            ci = sched_ref[1, g * _K + k]
            brow = batch_ref[0, pl.ds(ri * _T, _T)]
            bcol = batch_ref[0, pl.ds(ci * _T, _T)]
            wrow = jnp.zeros((_T,), jnp.float32)
            wcol = jnp.zeros((_T,), jnp.float32)
            for b in range(_NB):
                wrow = wrow + (brow == b).astype(jnp.float32) * inv_ref[b]
                wcol = wcol + (bcol == b).astype(jnp.float32) * inv_ref[b]

            x = pred_refs[k][...]
            log_p = jax.nn.log_sigmoid(x)
            p = jnp.exp(log_p)
            omp = 1.0 - p
            loss = -jnp.where(
                y_refs[k][...] != 0, omp * omp * log_p, p * p * (log_p - x)
            )

            eq = brow[:, None] == bcol[None, :]
            wmat = wrow[:, None] * wcol[None, :]
            v = jnp.where(eq, loss * wmat, 0.0)
            h = _T
            while h > 8:
                h //= 2
                v = v[:h] + v[h:]
            v = v[:, :128] + v[:, 128:]
            acc_ref[...] += v

    @pl.when(g == _G - 1)
    def _():
        out_ref[...] = jnp.sum(acc_ref[...]).reshape(1, 1)


def _make_schedule(batch):
    first = batch[:: _T]
    last = batch[_T - 1 :: _T]
    needed = (first[:, None] <= last[None, :]) & (first[None, :] <= last[:, None])
    flat = needed.reshape(-1)
    m = jnp.sum(flat.astype(jnp.int32))

    pos = jnp.cumsum(flat.astype(jnp.int32)) - 1
    tile_ids = jnp.arange(_NTILES, dtype=jnp.int32)
    perm = jnp.zeros(_NTILES, jnp.int32).at[
        jnp.where(flat, pos, _NTILES)
    ].set(tile_ids, mode="drop")
    idx = jnp.where(tile_ids < m, perm, perm[m - 1])
    sched = jnp.stack(
        [idx // _NT, idx % _NT, jnp.full((_NTILES,), m, dtype=jnp.int32)]
    )
    return sched


def kernel(y_seg_pred, y_seg, batch):
    batch = batch.astype(jnp.int32)
    sched = _make_schedule(batch)
    batch2d = batch.reshape(1, _N)

    def tile_spec(k):
        return pl.BlockSpec(
            (_T, _T), lambda g, s, k=k: (s[0, g * _K + k], s[1, g * _K + k])
        )

    total = pl.pallas_call(
        _focal_body,
        grid_spec=pltpu.PrefetchScalarGridSpec(
            num_scalar_prefetch=1,
            grid=(_G,),
            in_specs=[pl.BlockSpec((1, _N), lambda g, s: (0, 0))]
            + [tile_spec(k) for k in range(_K)]
            + [tile_spec(k) for k in range(_K)],
            out_specs=pl.BlockSpec((1, 1), lambda g, s: (0, 0)),
            scratch_shapes=[
                pltpu.SMEM((_NB,), jnp.float32),
                pltpu.VMEM((8, 128), jnp.float32),
            ],
        ),
        out_shape=jax.ShapeDtypeStruct((1, 1), jnp.float32),
    )(sched, batch2d, *([y_seg_pred] * _K), *([y_seg] * _K))
    batch_size = (batch[-1] + 1).astype(jnp.float32)
    return total[0, 0] / batch_size


# speedup vs baseline: 1.8822x; 1.0012x over previous
import jax
import jax.numpy as jnp
from jax.experimental import pallas as pl
from jax.experimental.pallas import tpu as pltpu

_N = 4096
_NB = 4
_T = 256
_NT = _N // _T
_K = 8
_NTILES = _NT * _NT
_G = _NTILES // _K


def _focal_body(sched_ref, batch_ref, *refs):
    pred_refs = refs[:_K]
    y_refs = refs[_K : 2 * _K]
    out_ref = refs[2 * _K]
    inv_ref = refs[2 * _K + 1]
    acc_ref = refs[2 * _K + 2]
    g = pl.program_id(0)
    m_valid = sched_ref[2, 0]

    @pl.when(g == 0)
    def _():
        acc_ref[...] = jnp.zeros_like(acc_ref)
        b_all = batch_ref[0, :]
        for b in range(_NB):
            cnt = jnp.sum((b_all == b).astype(jnp.float32))
            inv_ref[b] = 1.0 / jnp.maximum(cnt, 1.0)

    for k in range(_K):
        @pl.when(g * _K + k < m_valid)
        def _(k=k):
            ri = sched_ref[0, g * _K + k]
            ci = sched_ref[1, g * _K + k]
            brow = batch_ref[0, pl.ds(ri * _T, _T)]
            bcol = batch_ref[0, pl.ds(ci * _T, _T)]
            wrow = jnp.zeros((_T,), jnp.float32)
            wcol = jnp.zeros((_T,), jnp.float32)
            for b in range(_NB):
                wrow = wrow + (brow == b).astype(jnp.float32) * inv_ref[b]
                wcol = wcol + (bcol == b).astype(jnp.float32) * inv_ref[b]

            x = pred_refs[k][...]
            log_p = jax.nn.log_sigmoid(x)
            p = jnp.exp(log_p)
            omp = 1.0 - p
            loss = -jnp.where(
                y_refs[k][...] != 0, omp * omp * log_p, p * p * (log_p - x)
            )

            eq = brow[:, None] == bcol[None, :]
            wmat = wrow[:, None] * wcol[None, :]
            v = jnp.where(eq, loss * wmat, 0.0)
            h = _T
            while h > 8:
                h //= 2
                v = v[:h] + v[h:]
            v = v[:, :128] + v[:, 128:]
            acc_ref[...] += v

    @pl.when(g == _G - 1)
    def _():
        out_ref[...] = jnp.sum(acc_ref[...]).reshape(1, 1)


def _make_schedule(batch):
    first = batch[:: _T]
    last = batch[_T - 1 :: _T]
    needed = (first[:, None] <= last[None, :]) & (first[None, :] <= last[:, None])
    flat = needed.reshape(-1)
    m = jnp.sum(flat.astype(jnp.int32))

    pos = jnp.cumsum(flat.astype(jnp.int32)) - 1
    tile_ids = jnp.arange(_NTILES, dtype=jnp.int32)
    perm = jnp.zeros(_NTILES, jnp.int32).at[
        jnp.where(flat, pos, _NTILES)
    ].set(tile_ids, mode="drop")
    idx = jnp.where(tile_ids < m, perm, perm[m - 1])
    sched = jnp.stack(
        [idx // _NT, idx % _NT, jnp.full((_NTILES,), m, dtype=jnp.int32)]
    )
    return sched


def kernel(y_seg_pred, y_seg, batch):
    batch = batch.astype(jnp.int32)
    sched = _make_schedule(batch)
    batch2d = batch.reshape(1, _N)

    def tile_spec(k):
        return pl.BlockSpec(
            (_T, _T), lambda g, s, k=k: (s[0, g * _K + k], s[1, g * _K + k])
        )

    total = pl.pallas_call(
        _focal_body,
        grid_spec=pltpu.PrefetchScalarGridSpec(
            num_scalar_prefetch=1,
            grid=(_G,),
            in_specs=[pl.BlockSpec((1, _N), lambda g, s: (0, 0))]
            + [tile_spec(k) for k in range(_K)]
            + [tile_spec(k) for k in range(_K)],
            out_specs=pl.BlockSpec((1, 1), lambda g, s: (0, 0)),
            scratch_shapes=[
                pltpu.SMEM((_NB,), jnp.float32),
                pltpu.VMEM((8, 128), jnp.float32),
            ],
        ),
        out_shape=jax.ShapeDtypeStruct((1, 1), jnp.float32),
    )(sched, batch2d, *([y_seg_pred] * _K), *([y_seg] * _K))
    batch_size = (batch[-1] + 1).astype(jnp.float32)
    return total[0, 0] / batch_size
